# baseline (speedup 1.0000x reference)
import jax
import jax.numpy as jnp
from jax import lax
from jax.experimental import pallas as pl
from jax.experimental.pallas import tpu as pltpu
from jax.experimental.pallas import tpu_sc as plsc

_ROWS = 8192
_EMB = 1024
_NBUF = 2

_S_SIZES = (16,) * 11
_S_PER_TILE = sum(_S_SIZES)
_STREAM_ROWS = 30 * _S_PER_TILE

_P_SIZES = (208,) * 7
_P_PER_CORE = sum(_P_SIZES)

assert _STREAM_ROWS + 2 * _P_PER_CORE == _ROWS


def _ring(table_hbm, out_hbm, bufs, sis, sos, base, sizes):
    nchunk = len(sizes)
    offs = [sum(sizes[:i]) for i in range(nchunk)]
    ins = [
        pltpu.make_async_copy(
            table_hbm.at[pl.ds(base + offs[i], sizes[i])],
            bufs[i % _NBUF],
            sis[i % _NBUF],
        )
        for i in range(nchunk)
    ]
    outs = [
        pltpu.make_async_copy(
            bufs[i % _NBUF],
            out_hbm.at[pl.ds(base + offs[i], sizes[i])],
            sos[i % _NBUF],
        )
        for i in range(nchunk)
    ]
    for i in range(-(_NBUF - 1), nchunk):
        k = i + _NBUF - 1
        if 0 <= k < nchunk:
            if k >= _NBUF:
                outs[k - _NBUF].wait()
            ins[k].start()
        if i >= 0:
            ins[i].wait()
            outs[i].start()
    for i in range(max(0, nchunk - _NBUF), nchunk):
        outs[i].wait()


def _copy_body(table_hbm, out_hbm, *scratch):
    vbufs = scratch[:_NBUF]
    pbufs = scratch[_NBUF:2 * _NBUF]
    sis = scratch[2 * _NBUF:3 * _NBUF]
    sos = scratch[3 * _NBUF:]
    c = lax.axis_index("c")
    s = lax.axis_index("s")

    @pl.when(s == 0)
    def _spmem_path():
        base = _STREAM_ROWS + c * _P_PER_CORE
        _ring(table_hbm, out_hbm, pbufs, sis, sos, base, _P_SIZES)

    @pl.when(s != 0)
    def _stream_path():
        swid = (s - 1) * 2 + c
        base = swid * _S_PER_TILE
        _ring(table_hbm, out_hbm, vbufs, sis, sos, base, _S_SIZES)


def kernel(x, table):
    del x
    mesh = plsc.VectorSubcoreMesh(core_axis_name="c", subcore_axis_name="s")
    out = pl.kernel(
        _copy_body,
        out_type=jax.ShapeDtypeStruct((_ROWS, _EMB), jnp.float32),
        mesh=mesh,
        scratch_types=(
            [pltpu.VMEM((_S_SIZES[0], _EMB), jnp.float32)] * _NBUF
            + [pltpu.VMEM_SHARED((_P_SIZES[0], _EMB), jnp.float32)] * _NBUF
            + [pltpu.SemaphoreType.DMA] * (2 * _NBUF)
        ),
    )(table)
    return out[None]


# speedup vs baseline: 1.0352x; 1.0352x over previous
import jax
import jax.numpy as jnp
from jax import lax
from jax.experimental import pallas as pl
from jax.experimental.pallas import tpu as pltpu
from jax.experimental.pallas import tpu_sc as plsc

_ROWS = 8192
_EMB = 1024
_NBUF = 2

_S_SIZES = (32,) * 5
_S_PER_TILE = sum(_S_SIZES)
_STREAM_ROWS = 30 * _S_PER_TILE

_P_SIZES = (424,) * 4
_P_PER_CORE = sum(_P_SIZES)

assert _STREAM_ROWS + 2 * _P_PER_CORE == _ROWS


def _ring(table_hbm, out_hbm, bufs, sis, sos, base, sizes):
    nchunk = len(sizes)
    offs = [sum(sizes[:i]) for i in range(nchunk)]
    ins = [
        pltpu.make_async_copy(
            table_hbm.at[pl.ds(base + offs[i], sizes[i])],
            bufs[i % _NBUF],
            sis[i % _NBUF],
        )
        for i in range(nchunk)
    ]
    outs = [
        pltpu.make_async_copy(
            bufs[i % _NBUF],
            out_hbm.at[pl.ds(base + offs[i], sizes[i])],
            sos[i % _NBUF],
        )
        for i in range(nchunk)
    ]
    for i in range(-(_NBUF - 1), nchunk):
        k = i + _NBUF - 1
        if 0 <= k < nchunk:
            if k >= _NBUF:
                outs[k - _NBUF].wait()
            ins[k].start()
        if i >= 0:
            ins[i].wait()
            outs[i].start()
    for i in range(max(0, nchunk - _NBUF), nchunk):
        outs[i].wait()


def _copy_body(table_hbm, out_hbm, *scratch):
    vbufs = scratch[:_NBUF]
    pbufs = scratch[_NBUF:2 * _NBUF]
    sis = scratch[2 * _NBUF:3 * _NBUF]
    sos = scratch[3 * _NBUF:]
    c = lax.axis_index("c")
    s = lax.axis_index("s")

    @pl.when(s == 0)
    def _spmem_path():
        base = _STREAM_ROWS + c * _P_PER_CORE
        _ring(table_hbm, out_hbm, pbufs, sis, sos, base, _P_SIZES)

    @pl.when(s != 0)
    def _stream_path():
        swid = (s - 1) * 2 + c
        base = swid * _S_PER_TILE
        _ring(table_hbm, out_hbm, vbufs, sis, sos, base, _S_SIZES)


def kernel(x, table):
    del x
    mesh = plsc.VectorSubcoreMesh(core_axis_name="c", subcore_axis_name="s")
    out = pl.kernel(
        _copy_body,
        out_type=jax.ShapeDtypeStruct((_ROWS, _EMB), jnp.float32),
        mesh=mesh,
        scratch_types=(
            [pltpu.VMEM((_S_SIZES[0], _EMB), jnp.float32)] * _NBUF
            + [pltpu.VMEM_SHARED((_P_SIZES[0], _EMB), jnp.float32)] * _NBUF
            + [pltpu.SemaphoreType.DMA] * (2 * _NBUF)
        ),
    )(table)
    return out[None]
